# baseline (speedup 1.0000x reference)
import functools

import jax
import jax.numpy as jnp
from jax import lax
from jax.experimental import pallas as pl
from jax.experimental.pallas import tpu as pltpu
from jax.experimental.pallas import tpu_sc as plsc

_N = 10000
_E = 320000
_DIN = 128
_DH = 128
_F = 64
_KHOP = 10
_TEMP = 1.5
_EPS = 1e-5

_NC, _NS, _L = 2, 16, 16
_NW = _NC * _NS
_NPAD = 10240
_RPT = _NPAD // _NW
_RPS = _NPAD // _NS
_RC = 64
_C = 256
_ET = _E + _N
_CHUNKS = 42
_EPT = _CHUNKS * _C
_ETPAD = _NW * _EPT
_TRASH = _N

_mesh = plsc.VectorSubcoreMesh(core_axis_name="c", subcore_axis_name="s")
_f32 = jnp.float32


def _rsqrt16(d):
    u = d
    for _ in range(18):
        u = 0.5 * (u + d / u)
    return 1.0 / u


def _gelu(t):
    return 0.5 * t * (1.0 + lax.erf(t * 0.7071067811865476))


def _mlp_body(x_ref, w1, b1, w2, b2, w3, b3, w4, b4, o_ref):
    xb = x_ref[...]
    x0 = _gelu(jnp.dot(xb, w1[...], preferred_element_type=_f32) + b1[...])
    x1 = _gelu(jnp.dot(x0, w2[...], preferred_element_type=_f32) + b2[...] + x0)
    x2 = _gelu(jnp.dot(x1, w3[...], preferred_element_type=_f32) + b3[...] + x1)
    o_ref[...] = jnp.dot(x2, w4[...], preferred_element_type=_f32) + b4[...]


_BM = 512


def _tc_mlp(xp, w1, b1, w2, b2, w3, b3, w4, b4):
    def _fixed(r, c):
        return pl.BlockSpec((r, c), lambda i: (0, 0))

    return pl.pallas_call(
        _mlp_body,
        grid=(_NPAD // _BM,),
        in_specs=[
            pl.BlockSpec((_BM, _DIN), lambda i: (i, 0)),
            _fixed(_DIN, _DH), _fixed(1, _DH),
            _fixed(_DH, _DH), _fixed(1, _DH),
            _fixed(_DH, _DH), _fixed(1, _DH),
            _fixed(_DH, _F), _fixed(1, _F),
        ],
        out_specs=pl.BlockSpec((_BM, _F), lambda i: (i, 0)),
        out_shape=jax.ShapeDtypeStruct((_NPAD, _F), _f32),
    )(xp, w1, b1, w2, b2, w3, b3, w4, b4)


@functools.partial(
    pl.kernel, mesh=_mesh,
    compiler_params=pltpu.CompilerParams(use_tc_tiling_on_sc=False),
    out_type=jax.ShapeDtypeStruct((_NC, _NPAD, _L), _f32),
    scratch_types=[
        pltpu.VMEM((_CHUNKS, _C), jnp.int32),
        pltpu.VMEM((_C, _L), _f32),
        pltpu.VMEM((_C, _L), _f32),
        pltpu.VMEM_SHARED((_NPAD, _L), _f32),
    ],
)
def _sc_deg(row_hbm, out_hbm, row_v, ones_v, zbuf, acc):
    c = lax.axis_index("c")
    s = lax.axis_index("s")
    wid = s * _NC + c

    def fill(j, _):
        ones_v[j, :] = jnp.full((_L,), 1.0, _f32)
        zbuf[j, :] = jnp.zeros((_L,), _f32)
        return 0

    lax.fori_loop(0, _C, fill, 0)
    for t in range(_RPS // _C):
        pltpu.sync_copy(zbuf, acc.at[pl.ds(s * _RPS + t * _C, _C)])
    if _RPS % _C:
        pltpu.sync_copy(zbuf.at[pl.ds(0, _RPS % _C)],
                        acc.at[pl.ds(s * _RPS + (_RPS // _C) * _C, _RPS % _C)])
    pltpu.sync_copy(row_hbm.at[wid], row_v)
    plsc.subcore_barrier()

    def body(j, _):
        pltpu.sync_copy(ones_v, acc.at[row_v.at[j]], add=True)
        return 0

    lax.fori_loop(0, _CHUNKS, body, 0)
    plsc.subcore_barrier()
    pltpu.sync_copy(acc.at[pl.ds(s * _RPS, _RPS)],
                    out_hbm.at[c, pl.ds(s * _RPS, _RPS)])


@functools.partial(
    pl.kernel, mesh=_mesh,
    compiler_params=pltpu.CompilerParams(use_tc_tiling_on_sc=False),
    out_type=(jax.ShapeDtypeStruct((_NPAD, _F), _f32),
              jax.ShapeDtypeStruct((_NPAD, _F), _f32),
              jax.ShapeDtypeStruct((_NPAD, _F), _f32)),
    scratch_types=[
        pltpu.VMEM((_RC, _L), _f32),
        pltpu.VMEM((_RC, _L), _f32),
        pltpu.VMEM((_RC, _F), _f32),
        pltpu.VMEM((_RC, _F), _f32),
        pltpu.VMEM((_RC, _F), _f32),
        pltpu.VMEM((_RC, _F), _f32),
        pltpu.VMEM((_L,), _f32),
    ],
)
def _sc_init(degp, h_hbm, a_hbm, dinv_out, w_out, xp_out,
             d0b, d1b, hb, dob, wb, xpb, avb):
    c = lax.axis_index("c")
    s = lax.axis_index("s")
    wid = s * _NC + c
    pltpu.sync_copy(a_hbm, avb)
    a = avb[...]
    for t in range(_RPT // _RC):
        base = wid * _RPT + t * _RC
        pltpu.sync_copy(degp.at[0, pl.ds(base, _RC)], d0b)
        pltpu.sync_copy(degp.at[1, pl.ds(base, _RC)], d1b)
        pltpu.sync_copy(h_hbm.at[pl.ds(base, _RC)], hb)

        def rowbody(j, _):
            deg = d0b[j, :] + d1b[j, :]
            deg = jnp.maximum(deg, jnp.ones_like(deg))
            z = _rsqrt16(deg)
            for fb in range(_F // _L):
                sl = pl.ds(fb * _L, _L)
                hv = hb[j, sl]
                dob[j, sl] = z
                wb[j, sl] = z * hv
                xpb[j, sl] = a * hv
            return 0

        lax.fori_loop(0, _RC, rowbody, 0)
        pltpu.sync_copy(dob, dinv_out.at[pl.ds(base, _RC)])
        pltpu.sync_copy(wb, w_out.at[pl.ds(base, _RC)])
        pltpu.sync_copy(xpb, xp_out.at[pl.ds(base, _RC)])


@functools.partial(
    pl.kernel, mesh=_mesh,
    compiler_params=pltpu.CompilerParams(use_tc_tiling_on_sc=False),
    out_type=jax.ShapeDtypeStruct((_NC, _NPAD, _F), _f32),
    scratch_types=[
        pltpu.VMEM((_CHUNKS, _C), jnp.int32),
        pltpu.VMEM((_CHUNKS, _C), jnp.int32),
        [pltpu.VMEM((_C, _F), _f32)] * 3,
        [pltpu.SemaphoreType.DMA] * 3,
        pltpu.VMEM_SHARED((_NPAD, _F), _f32),
    ],
)
def _sc_hop(w_hbm, col_hbm, row_hbm, out_hbm,
            col_v, row_v, bufs, gs, acc):
    c = lax.axis_index("c")
    s = lax.axis_index("s")
    wid = s * _NC + c

    def zfill(j, _):
        for fb in range(_F // _L):
            bufs[0][j, pl.ds(fb * _L, _L)] = jnp.zeros((_L,), _f32)
        return 0

    lax.fori_loop(0, _C, zfill, 0)
    for t in range(_RPS // _C):
        pltpu.sync_copy(bufs[0], acc.at[pl.ds(s * _RPS + t * _C, _C)])
    if _RPS % _C:
        pltpu.sync_copy(bufs[0].at[pl.ds(0, _RPS % _C)],
                        acc.at[pl.ds(s * _RPS + (_RPS // _C) * _C, _RPS % _C)])
    pltpu.sync_copy(col_hbm.at[wid], col_v)
    pltpu.sync_copy(row_hbm.at[wid], row_v)
    plsc.subcore_barrier()

    for b in range(3):
        pltpu.async_copy(w_hbm.at[col_v.at[b]], bufs[b], gs[b])

    def body(i, _):
        for b in range(3):
            j = 3 * i + b
            pltpu.make_async_copy(w_hbm.at[col_v.at[j]], bufs[b],
                                  gs[b]).wait()
            pltpu.sync_copy(bufs[b], acc.at[row_v.at[j]], add=True)
            pltpu.async_copy(w_hbm.at[col_v.at[j + 3]], bufs[b], gs[b])
        return 0

    lax.fori_loop(0, _CHUNKS // 3 - 1, body, 0)
    for b in range(3):
        j = _CHUNKS - 3 + b
        pltpu.make_async_copy(w_hbm.at[col_v.at[j]], bufs[b], gs[b]).wait()
        pltpu.sync_copy(bufs[b], acc.at[row_v.at[j]], add=True)
    plsc.subcore_barrier()
    pltpu.sync_copy(acc.at[pl.ds(s * _RPS, _RPS)],
                    out_hbm.at[c, pl.ds(s * _RPS, _RPS)])


@functools.partial(
    pl.kernel, mesh=_mesh,
    compiler_params=pltpu.CompilerParams(use_tc_tiling_on_sc=False),
    out_type=(jax.ShapeDtypeStruct((_NPAD, _F), _f32),
              jax.ShapeDtypeStruct((_NPAD, _F), _f32)),
    scratch_types=[
        pltpu.VMEM((_RC, _F), _f32),
        pltpu.VMEM((_RC, _F), _f32),
        pltpu.VMEM((_RC, _F), _f32),
        pltpu.VMEM((_RC, _F), _f32),
        pltpu.VMEM((_RC, _F), _f32),
        pltpu.VMEM((_L,), _f32),
    ],
)
def _sc_combine(p_hbm, dinv_hbm, xp_hbm, a_hbm, w_out, xp_out,
                p0b, p1b, db, xb, wb, avb):
    c = lax.axis_index("c")
    s = lax.axis_index("s")
    wid = s * _NC + c
    pltpu.sync_copy(a_hbm, avb)
    a = avb[...]
    for t in range(_RPT // _RC):
        base = wid * _RPT + t * _RC
        pltpu.sync_copy(p_hbm.at[0, pl.ds(base, _RC)], p0b)
        pltpu.sync_copy(p_hbm.at[1, pl.ds(base, _RC)], p1b)
        pltpu.sync_copy(dinv_hbm.at[pl.ds(base, _RC)], db)
        pltpu.sync_copy(xp_hbm.at[pl.ds(base, _RC)], xb)

        def rowbody(j, _):
            for fb in range(_F // _L):
                sl = pl.ds(fb * _L, _L)
                s16 = p0b[j, sl] + p1b[j, sl]
                dv = db[j, sl]
                dsv = dv * s16
                wb[j, sl] = dv * dsv
                xb[j, sl] = xb[j, sl] + a * dsv
            return 0

        lax.fori_loop(0, _RC, rowbody, 0)
        pltpu.sync_copy(wb, w_out.at[pl.ds(base, _RC)])
        pltpu.sync_copy(xb, xp_out.at[pl.ds(base, _RC)])


def kernel(x, edge_index, W1, b1, W2, b2, W3, b3, W4, b4,
           g1, be1, m1, v1, g2, be2, m2, v2, g3, be3, m3, v3, alpha):
    s1 = g1 * lax.rsqrt(v1 + _EPS)
    s2 = g2 * lax.rsqrt(v2 + _EPS)
    s3 = g3 * lax.rsqrt(v3 + _EPS)
    w1f = W1 * s1[None, :]
    w2f = W2 * s2[None, :]
    w3f = W3 * s3[None, :]
    b1f = (b1 * s1 + be1 - m1 * s1).reshape(1, _DH)
    b2f = (b2 * s2 + be2 - m2 * s2).reshape(1, _DH)
    b3f = (b3 * s3 + be3 - m3 * s3).reshape(1, _DH)

    xp = jnp.pad(x, ((0, _NPAD - _N), (0, 0)))
    h = _tc_mlp(xp, w1f, b1f, w2f, b2f, w3f, b3f, W4, b4.reshape(1, _F))

    loops = jnp.arange(_N, dtype=edge_index.dtype)
    row = jnp.concatenate([edge_index[0], loops,
                           jnp.full((_ETPAD - _ET,), _TRASH, edge_index.dtype)])
    col = jnp.concatenate([edge_index[1], loops,
                           jnp.zeros((_ETPAD - _ET,), edge_index.dtype)])
    row3 = row.reshape(_NW, _CHUNKS, _C)
    col3 = col.reshape(_NW, _CHUNKS, _C)

    avecs = jnp.broadcast_to((alpha / _TEMP)[:, None], (_KHOP + 1, _L))

    degp = _sc_deg(row3)
    dinv64, w, xprop = _sc_init(degp, h, avecs[0])
    for k in range(1, _KHOP + 1):
        p = _sc_hop(w, col3, row3)
        w, xprop = _sc_combine(p, dinv64, xprop, avecs[k])
    return xprop[:_N]


# speedup vs baseline: 1.0000x; 1.0000x over previous
import functools

import jax
import jax.numpy as jnp
from jax import lax
from jax.experimental import pallas as pl
from jax.experimental.pallas import tpu as pltpu
from jax.experimental.pallas import tpu_sc as plsc

_N = 10000
_E = 320000
_DIN = 128
_DH = 128
_F = 64
_KHOP = 10
_TEMP = 1.5
_EPS = 1e-5

_NC, _NS, _L = 2, 16, 16
_NW = _NC * _NS
_NPAD = 10240
_RPT = _NPAD // _NW
_RPS = _NPAD // _NS
_RC = 64
_C = 256
_ET = _E + _N
_CHUNKS = 42
_EPT = _CHUNKS * _C
_ETPAD = _NW * _EPT
_TRASH = _N

_mesh = plsc.VectorSubcoreMesh(core_axis_name="c", subcore_axis_name="s")
_f32 = jnp.float32


def _rsqrt16(d):
    u = d
    for _ in range(18):
        u = 0.5 * (u + d / u)
    return 1.0 / u


def _gelu(t):
    return 0.5 * t * (1.0 + lax.erf(t * 0.7071067811865476))


def _mlp_body(x_ref, w1, b1, w2, b2, w3, b3, w4, b4, o_ref):
    xb = x_ref[...]
    x0 = _gelu(jnp.dot(xb, w1[...], preferred_element_type=_f32) + b1[...])
    x1 = _gelu(jnp.dot(x0, w2[...], preferred_element_type=_f32) + b2[...] + x0)
    x2 = _gelu(jnp.dot(x1, w3[...], preferred_element_type=_f32) + b3[...] + x1)
    o_ref[...] = jnp.dot(x2, w4[...], preferred_element_type=_f32) + b4[...]


_BM = 512


def _tc_mlp(xp, w1, b1, w2, b2, w3, b3, w4, b4):
    def _fixed(r, c):
        return pl.BlockSpec((r, c), lambda i: (0, 0))

    return pl.pallas_call(
        _mlp_body,
        grid=(_NPAD // _BM,),
        in_specs=[
            pl.BlockSpec((_BM, _DIN), lambda i: (i, 0)),
            _fixed(_DIN, _DH), _fixed(1, _DH),
            _fixed(_DH, _DH), _fixed(1, _DH),
            _fixed(_DH, _DH), _fixed(1, _DH),
            _fixed(_DH, _F), _fixed(1, _F),
        ],
        out_specs=pl.BlockSpec((_BM, _F), lambda i: (i, 0)),
        out_shape=jax.ShapeDtypeStruct((_NPAD, _F), _f32),
    )(xp, w1, b1, w2, b2, w3, b3, w4, b4)


@functools.partial(
    pl.kernel, mesh=_mesh,
    compiler_params=pltpu.CompilerParams(use_tc_tiling_on_sc=False),
    out_type=jax.ShapeDtypeStruct((_NC, _NPAD, _L), _f32),
    scratch_types=[
        pltpu.VMEM((_CHUNKS, _C), jnp.int32),
        pltpu.VMEM((_C, _L), _f32),
        pltpu.VMEM((_C, _L), _f32),
        pltpu.VMEM_SHARED((_NPAD, _L), _f32),
    ],
)
def _sc_deg(row_hbm, out_hbm, row_v, ones_v, zbuf, acc):
    c = lax.axis_index("c")
    s = lax.axis_index("s")
    wid = s * _NC + c

    def fill(j, _):
        ones_v[j, :] = jnp.full((_L,), 1.0, _f32)
        zbuf[j, :] = jnp.zeros((_L,), _f32)
        return 0

    lax.fori_loop(0, _C, fill, 0)
    for t in range(_RPS // _C):
        pltpu.sync_copy(zbuf, acc.at[pl.ds(s * _RPS + t * _C, _C)])
    if _RPS % _C:
        pltpu.sync_copy(zbuf.at[pl.ds(0, _RPS % _C)],
                        acc.at[pl.ds(s * _RPS + (_RPS // _C) * _C, _RPS % _C)])
    pltpu.sync_copy(row_hbm.at[wid], row_v)
    plsc.subcore_barrier()

    def body(j, _):
        pltpu.sync_copy(ones_v, acc.at[row_v.at[j]], add=True)
        return 0

    lax.fori_loop(0, _CHUNKS, body, 0)
    plsc.subcore_barrier()
    pltpu.sync_copy(acc.at[pl.ds(s * _RPS, _RPS)],
                    out_hbm.at[c, pl.ds(s * _RPS, _RPS)])


@functools.partial(
    pl.kernel, mesh=_mesh,
    compiler_params=pltpu.CompilerParams(use_tc_tiling_on_sc=False),
    out_type=(jax.ShapeDtypeStruct((_NPAD, _F), _f32),
              jax.ShapeDtypeStruct((_NPAD, _F), _f32),
              jax.ShapeDtypeStruct((_NPAD, _F), _f32)),
    scratch_types=[
        pltpu.VMEM((_RC, _L), _f32),
        pltpu.VMEM((_RC, _L), _f32),
        pltpu.VMEM((_RC, _F), _f32),
        pltpu.VMEM((_RC, _F), _f32),
        pltpu.VMEM((_RC, _F), _f32),
        pltpu.VMEM((_RC, _F), _f32),
        pltpu.VMEM((_L,), _f32),
    ],
)
def _sc_init(degp, h_hbm, a_hbm, dinv_out, w_out, xp_out,
             d0b, d1b, hb, dob, wb, xpb, avb):
    c = lax.axis_index("c")
    s = lax.axis_index("s")
    wid = s * _NC + c
    pltpu.sync_copy(a_hbm, avb)
    a = avb[...]
    for t in range(_RPT // _RC):
        base = wid * _RPT + t * _RC
        pltpu.sync_copy(degp.at[0, pl.ds(base, _RC)], d0b)
        pltpu.sync_copy(degp.at[1, pl.ds(base, _RC)], d1b)
        pltpu.sync_copy(h_hbm.at[pl.ds(base, _RC)], hb)

        def rowbody(j, _):
            deg = d0b[j, :] + d1b[j, :]
            deg = jnp.maximum(deg, jnp.ones_like(deg))
            z = _rsqrt16(deg)
            for fb in range(_F // _L):
                sl = pl.ds(fb * _L, _L)
                hv = hb[j, sl]
                dob[j, sl] = z
                wb[j, sl] = z * hv
                xpb[j, sl] = a * hv
            return 0

        lax.fori_loop(0, _RC, rowbody, 0)
        pltpu.sync_copy(dob, dinv_out.at[pl.ds(base, _RC)])
        pltpu.sync_copy(wb, w_out.at[pl.ds(base, _RC)])
        pltpu.sync_copy(xpb, xp_out.at[pl.ds(base, _RC)])


@functools.partial(
    pl.kernel, mesh=_mesh,
    compiler_params=pltpu.CompilerParams(use_tc_tiling_on_sc=False),
    out_type=jax.ShapeDtypeStruct((_NC, _NPAD, _F), _f32),
    scratch_types=[
        pltpu.VMEM((_CHUNKS, _C), jnp.int32),
        pltpu.VMEM((_CHUNKS, _C), jnp.int32),
        pltpu.VMEM((_C, _F), _f32),
        pltpu.VMEM((_C, _F), _f32),
        pltpu.VMEM((_C, _F), _f32),
        pltpu.SemaphoreType.DMA,
        pltpu.SemaphoreType.DMA,
        pltpu.SemaphoreType.DMA,
        pltpu.VMEM_SHARED((_NPAD, _F), _f32),
    ],
)
def _sc_hop(w_hbm, col_hbm, row_hbm, out_hbm,
            col_v, row_v, buf0, buf1, buf2, g0, g1, g2, acc):
    c = lax.axis_index("c")
    s = lax.axis_index("s")
    wid = s * _NC + c
    bufs = (buf0, buf1, buf2)
    gs = (g0, g1, g2)

    def zfill(j, _):
        for fb in range(_F // _L):
            bufs[0][j, pl.ds(fb * _L, _L)] = jnp.zeros((_L,), _f32)
        return 0

    lax.fori_loop(0, _C, zfill, 0)
    for t in range(_RPS // _C):
        pltpu.sync_copy(bufs[0], acc.at[pl.ds(s * _RPS + t * _C, _C)])
    if _RPS % _C:
        pltpu.sync_copy(bufs[0].at[pl.ds(0, _RPS % _C)],
                        acc.at[pl.ds(s * _RPS + (_RPS // _C) * _C, _RPS % _C)])
    pltpu.sync_copy(col_hbm.at[wid], col_v)
    pltpu.sync_copy(row_hbm.at[wid], row_v)
    plsc.subcore_barrier()

    for b in range(3):
        pltpu.async_copy(w_hbm.at[col_v.at[b]], bufs[b], gs[b])

    def body(i, _):
        for b in range(3):
            j = 3 * i + b
            pltpu.make_async_copy(w_hbm.at[col_v.at[j]], bufs[b],
                                  gs[b]).wait()
            pltpu.sync_copy(bufs[b], acc.at[row_v.at[j]], add=True)
            pltpu.async_copy(w_hbm.at[col_v.at[j + 3]], bufs[b], gs[b])
        return 0

    lax.fori_loop(0, _CHUNKS // 3 - 1, body, 0)
    for b in range(3):
        j = _CHUNKS - 3 + b
        pltpu.make_async_copy(w_hbm.at[col_v.at[j]], bufs[b], gs[b]).wait()
        pltpu.sync_copy(bufs[b], acc.at[row_v.at[j]], add=True)
    plsc.subcore_barrier()
    pltpu.sync_copy(acc.at[pl.ds(s * _RPS, _RPS)],
                    out_hbm.at[c, pl.ds(s * _RPS, _RPS)])


@functools.partial(
    pl.kernel, mesh=_mesh,
    compiler_params=pltpu.CompilerParams(use_tc_tiling_on_sc=False),
    out_type=(jax.ShapeDtypeStruct((_NPAD, _F), _f32),
              jax.ShapeDtypeStruct((_NPAD, _F), _f32)),
    scratch_types=[
        pltpu.VMEM((_RC, _F), _f32),
        pltpu.VMEM((_RC, _F), _f32),
        pltpu.VMEM((_RC, _F), _f32),
        pltpu.VMEM((_RC, _F), _f32),
        pltpu.VMEM((_RC, _F), _f32),
        pltpu.VMEM((_L,), _f32),
    ],
)
def _sc_combine(p_hbm, dinv_hbm, xp_hbm, a_hbm, w_out, xp_out,
                p0b, p1b, db, xb, wb, avb):
    c = lax.axis_index("c")
    s = lax.axis_index("s")
    wid = s * _NC + c
    pltpu.sync_copy(a_hbm, avb)
    a = avb[...]
    for t in range(_RPT // _RC):
        base = wid * _RPT + t * _RC
        pltpu.sync_copy(p_hbm.at[0, pl.ds(base, _RC)], p0b)
        pltpu.sync_copy(p_hbm.at[1, pl.ds(base, _RC)], p1b)
        pltpu.sync_copy(dinv_hbm.at[pl.ds(base, _RC)], db)
        pltpu.sync_copy(xp_hbm.at[pl.ds(base, _RC)], xb)

        def rowbody(j, _):
            for fb in range(_F // _L):
                sl = pl.ds(fb * _L, _L)
                s16 = p0b[j, sl] + p1b[j, sl]
                dv = db[j, sl]
                dsv = dv * s16
                wb[j, sl] = dv * dsv
                xb[j, sl] = xb[j, sl] + a * dsv
            return 0

        lax.fori_loop(0, _RC, rowbody, 0)
        pltpu.sync_copy(wb, w_out.at[pl.ds(base, _RC)])
        pltpu.sync_copy(xb, xp_out.at[pl.ds(base, _RC)])


def kernel(x, edge_index, W1, b1, W2, b2, W3, b3, W4, b4,
           g1, be1, m1, v1, g2, be2, m2, v2, g3, be3, m3, v3, alpha):
    s1 = g1 * lax.rsqrt(v1 + _EPS)
    s2 = g2 * lax.rsqrt(v2 + _EPS)
    s3 = g3 * lax.rsqrt(v3 + _EPS)
    w1f = W1 * s1[None, :]
    w2f = W2 * s2[None, :]
    w3f = W3 * s3[None, :]
    b1f = (b1 * s1 + be1 - m1 * s1).reshape(1, _DH)
    b2f = (b2 * s2 + be2 - m2 * s2).reshape(1, _DH)
    b3f = (b3 * s3 + be3 - m3 * s3).reshape(1, _DH)

    xp = jnp.pad(x, ((0, _NPAD - _N), (0, 0)))
    h = _tc_mlp(xp, w1f, b1f, w2f, b2f, w3f, b3f, W4, b4.reshape(1, _F))

    loops = jnp.arange(_N, dtype=edge_index.dtype)
    row = jnp.concatenate([edge_index[0], loops,
                           jnp.full((_ETPAD - _ET,), _TRASH, edge_index.dtype)])
    col = jnp.concatenate([edge_index[1], loops,
                           jnp.zeros((_ETPAD - _ET,), edge_index.dtype)])
    row3 = row.reshape(_NW, _CHUNKS, _C)
    col3 = col.reshape(_NW, _CHUNKS, _C)

    avecs = jnp.broadcast_to((alpha / _TEMP)[:, None], (_KHOP + 1, _L))

    degp = _sc_deg(row3)
    dinv64, w, xprop = _sc_init(degp, h, avecs[0])
    for k in range(1, _KHOP + 1):
        p = _sc_hop(w, col3, row3)
        w, xprop = _sc_combine(p, dinv64, xprop, avecs[k])
    return xprop[:_N]


# speedup vs baseline: 4.1597x; 4.1596x over previous
import functools

import jax
import jax.numpy as jnp
from jax import lax
from jax.experimental import pallas as pl
from jax.experimental.pallas import tpu as pltpu
from jax.experimental.pallas import tpu_sc as plsc

_N = 10000
_E = 320000
_DIN = 128
_DH = 128
_F = 64
_KHOP = 10
_TEMP = 1.5
_EPS = 1e-5

_NC, _NS, _L = 2, 16, 16
_NW = _NC * _NS
_NPAD = 10240
_RPT = _NPAD // _NW
_RPS = _NPAD // _NS
_RC = 64
_C = 256
_ET = _E + _N
_CHUNKS = 42
_EPT = _CHUNKS * _C
_ETPAD = _NW * _EPT
_TRASH = _N

_mesh = plsc.VectorSubcoreMesh(core_axis_name="c", subcore_axis_name="s")
_f32 = jnp.float32


def _rsqrt16(d):
    u = d
    for _ in range(18):
        u = 0.5 * (u + d / u)
    return 1.0 / u


def _gelu(t):
    return 0.5 * t * (1.0 + lax.erf(t * 0.7071067811865476))


def _mlp_body(x_ref, w1, b1, w2, b2, w3, b3, w4, b4, o_ref):
    xb = x_ref[...]
    x0 = _gelu(jnp.dot(xb, w1[...], preferred_element_type=_f32) + b1[...])
    x1 = _gelu(jnp.dot(x0, w2[...], preferred_element_type=_f32) + b2[...] + x0)
    x2 = _gelu(jnp.dot(x1, w3[...], preferred_element_type=_f32) + b3[...] + x1)
    o_ref[...] = jnp.dot(x2, w4[...], preferred_element_type=_f32) + b4[...]


_BM = 512


def _tc_mlp(xp, w1, b1, w2, b2, w3, b3, w4, b4):
    def _fixed(r, c):
        return pl.BlockSpec((r, c), lambda i: (0, 0))

    return pl.pallas_call(
        _mlp_body,
        grid=(_NPAD // _BM,),
        in_specs=[
            pl.BlockSpec((_BM, _DIN), lambda i: (i, 0)),
            _fixed(_DIN, _DH), _fixed(1, _DH),
            _fixed(_DH, _DH), _fixed(1, _DH),
            _fixed(_DH, _DH), _fixed(1, _DH),
            _fixed(_DH, _F), _fixed(1, _F),
        ],
        out_specs=pl.BlockSpec((_BM, _F), lambda i: (i, 0)),
        out_shape=jax.ShapeDtypeStruct((_NPAD, _F), _f32),
    )(xp, w1, b1, w2, b2, w3, b3, w4, b4)


@functools.partial(
    pl.kernel, mesh=_mesh,
    compiler_params=pltpu.CompilerParams(use_tc_tiling_on_sc=False),
    out_type=jax.ShapeDtypeStruct((_NC, _NPAD, _L), _f32),
    scratch_types=[
        pltpu.VMEM((_CHUNKS, _C), jnp.int32),
        pltpu.VMEM((_C, _L), _f32),
        pltpu.VMEM((_C, _L), _f32),
        pltpu.VMEM_SHARED((_NPAD, _L), _f32),
    ],
)
def _sc_deg(row_hbm, out_hbm, row_v, ones_v, zbuf, acc):
    c = lax.axis_index("c")
    s = lax.axis_index("s")
    wid = s * _NC + c

    def fill(j, _):
        ones_v[j, :] = jnp.full((_L,), 1.0, _f32)
        zbuf[j, :] = jnp.zeros((_L,), _f32)
        return 0

    lax.fori_loop(0, _C, fill, 0)
    for t in range(_RPS // _C):
        pltpu.sync_copy(zbuf, acc.at[pl.ds(s * _RPS + t * _C, _C)])
    if _RPS % _C:
        pltpu.sync_copy(zbuf.at[pl.ds(0, _RPS % _C)],
                        acc.at[pl.ds(s * _RPS + (_RPS // _C) * _C, _RPS % _C)])
    pltpu.sync_copy(row_hbm.at[wid], row_v)
    plsc.subcore_barrier()

    def body(j, _):
        pltpu.sync_copy(ones_v, acc.at[row_v.at[j]], add=True)
        return 0

    lax.fori_loop(0, _CHUNKS, body, 0)
    plsc.subcore_barrier()
    pltpu.sync_copy(acc.at[pl.ds(s * _RPS, _RPS)],
                    out_hbm.at[c, pl.ds(s * _RPS, _RPS)])


@functools.partial(
    pl.kernel, mesh=_mesh,
    compiler_params=pltpu.CompilerParams(use_tc_tiling_on_sc=False),
    out_type=(jax.ShapeDtypeStruct((_NPAD, _F), _f32),
              jax.ShapeDtypeStruct((_NPAD, _F), _f32),
              jax.ShapeDtypeStruct((_NPAD, _F), _f32)),
    scratch_types=[
        pltpu.VMEM((_RC, _L), _f32),
        pltpu.VMEM((_RC, _L), _f32),
        pltpu.VMEM((_RC, _F), _f32),
        pltpu.VMEM((_RC, _F), _f32),
        pltpu.VMEM((_RC, _F), _f32),
        pltpu.VMEM((_RC, _F), _f32),
        pltpu.VMEM((_L,), _f32),
    ],
)
def _sc_init(degp, h_hbm, a_hbm, dinv_out, w_out, xp_out,
             d0b, d1b, hb, dob, wb, xpb, avb):
    c = lax.axis_index("c")
    s = lax.axis_index("s")
    wid = s * _NC + c
    pltpu.sync_copy(a_hbm, avb)
    a = avb[...]
    for t in range(_RPT // _RC):
        base = wid * _RPT + t * _RC
        pltpu.sync_copy(degp.at[0, pl.ds(base, _RC)], d0b)
        pltpu.sync_copy(degp.at[1, pl.ds(base, _RC)], d1b)
        pltpu.sync_copy(h_hbm.at[pl.ds(base, _RC)], hb)

        def rowbody(j, _):
            deg = d0b[j, :] + d1b[j, :]
            deg = jnp.maximum(deg, jnp.ones_like(deg))
            z = _rsqrt16(deg)
            for fb in range(_F // _L):
                sl = pl.ds(fb * _L, _L)
                hv = hb[j, sl]
                dob[j, sl] = z
                wb[j, sl] = z * hv
                xpb[j, sl] = a * hv
            return 0

        lax.fori_loop(0, _RC, rowbody, 0)
        pltpu.sync_copy(dob, dinv_out.at[pl.ds(base, _RC)])
        pltpu.sync_copy(wb, w_out.at[pl.ds(base, _RC)])
        pltpu.sync_copy(xpb, xp_out.at[pl.ds(base, _RC)])


@functools.partial(
    pl.kernel, mesh=_mesh,
    compiler_params=pltpu.CompilerParams(use_tc_tiling_on_sc=False),
    out_type=jax.ShapeDtypeStruct((_NC, _NPAD, _F), _f32),
    scratch_types=[
        pltpu.VMEM((_CHUNKS, _C), jnp.int32),
        pltpu.VMEM((_CHUNKS, _C), jnp.int32),
        pltpu.VMEM((_C, _F), _f32),
        pltpu.VMEM((_C, _F), _f32),
        pltpu.VMEM((_C, _F), _f32),
        pltpu.SemaphoreType.DMA,
        pltpu.SemaphoreType.DMA,
        pltpu.SemaphoreType.DMA,
        pltpu.VMEM_SHARED((_NPAD, _F), _f32),
    ],
)
def _sc_hop(w_hbm, col_hbm, row_hbm, out_hbm,
            col_v, row_v, buf0, buf1, buf2, g0, g1, g2, acc):
    c = lax.axis_index("c")
    s = lax.axis_index("s")
    wid = s * _NC + c
    bufs = (buf0, buf1, buf2)
    gs = (g0, g1, g2)

    def zfill(j, _):
        for fb in range(_F // _L):
            bufs[0][j, pl.ds(fb * _L, _L)] = jnp.zeros((_L,), _f32)
        return 0

    lax.fori_loop(0, _C, zfill, 0)
    for t in range(_RPS // _C):
        pltpu.sync_copy(bufs[0], acc.at[pl.ds(s * _RPS + t * _C, _C)])
    if _RPS % _C:
        pltpu.sync_copy(bufs[0].at[pl.ds(0, _RPS % _C)],
                        acc.at[pl.ds(s * _RPS + (_RPS // _C) * _C, _RPS % _C)])
    pltpu.sync_copy(col_hbm.at[wid], col_v)
    pltpu.sync_copy(row_hbm.at[wid], row_v)
    plsc.subcore_barrier()

    for b in range(3):
        pltpu.async_copy(w_hbm.at[col_v.at[b]], bufs[b], gs[b])

    def body(i, _):
        for b in range(3):
            j = 3 * i + b
            pltpu.make_async_copy(w_hbm.at[col_v.at[j]], bufs[b],
                                  gs[b]).wait()
            pltpu.sync_copy(bufs[b], acc.at[row_v.at[j]], add=True)
            pltpu.async_copy(w_hbm.at[col_v.at[j + 3]], bufs[b], gs[b])
        return 0

    lax.fori_loop(0, _CHUNKS // 3 - 1, body, 0)
    for b in range(3):
        j = _CHUNKS - 3 + b
        pltpu.make_async_copy(w_hbm.at[col_v.at[j]], bufs[b], gs[b]).wait()
        pltpu.sync_copy(bufs[b], acc.at[row_v.at[j]], add=True)
    plsc.subcore_barrier()
    pltpu.sync_copy(acc.at[pl.ds(s * _RPS, _RPS)],
                    out_hbm.at[c, pl.ds(s * _RPS, _RPS)])


@functools.partial(
    pl.kernel, mesh=_mesh,
    compiler_params=pltpu.CompilerParams(use_tc_tiling_on_sc=False),
    out_type=(jax.ShapeDtypeStruct((_NPAD, _F), _f32),
              jax.ShapeDtypeStruct((_NPAD, _F), _f32)),
    scratch_types=[
        pltpu.VMEM((_RC, _F), _f32),
        pltpu.VMEM((_RC, _F), _f32),
        pltpu.VMEM((_RC, _F), _f32),
        pltpu.VMEM((_RC, _F), _f32),
        pltpu.VMEM((_RC, _F), _f32),
        pltpu.VMEM((_L,), _f32),
    ],
)
def _sc_combine(p_hbm, dinv_hbm, xp_hbm, a_hbm, w_out, xp_out,
                p0b, p1b, db, xb, wb, avb):
    c = lax.axis_index("c")
    s = lax.axis_index("s")
    wid = s * _NC + c
    pltpu.sync_copy(a_hbm, avb)
    a = avb[...]
    for t in range(_RPT // _RC):
        base = wid * _RPT + t * _RC
        pltpu.sync_copy(p_hbm.at[0, pl.ds(base, _RC)], p0b)
        pltpu.sync_copy(p_hbm.at[1, pl.ds(base, _RC)], p1b)
        pltpu.sync_copy(dinv_hbm.at[pl.ds(base, _RC)], db)
        pltpu.sync_copy(xp_hbm.at[pl.ds(base, _RC)], xb)

        def rowbody(j, _):
            for fb in range(_F // _L):
                sl = pl.ds(fb * _L, _L)
                s16 = p0b[j, sl] + p1b[j, sl]
                dv = db[j, sl]
                dsv = dv * s16
                wb[j, sl] = dv * dsv
                xb[j, sl] = xb[j, sl] + a * dsv
            return 0

        lax.fori_loop(0, _RC, rowbody, 0)
        pltpu.sync_copy(wb, w_out.at[pl.ds(base, _RC)])
        pltpu.sync_copy(xb, xp_out.at[pl.ds(base, _RC)])


def kernel(x, edge_index, W1, b1, W2, b2, W3, b3, W4, b4,
           g1, be1, m1, v1, g2, be2, m2, v2, g3, be3, m3, v3, alpha):
    s1 = g1 * lax.rsqrt(v1 + _EPS)
    s2 = g2 * lax.rsqrt(v2 + _EPS)
    s3 = g3 * lax.rsqrt(v3 + _EPS)
    w1f = W1 * s1[None, :]
    w2f = W2 * s2[None, :]
    w3f = W3 * s3[None, :]
    b1f = (b1 * s1 + be1 - m1 * s1).reshape(1, _DH)
    b2f = (b2 * s2 + be2 - m2 * s2).reshape(1, _DH)
    b3f = (b3 * s3 + be3 - m3 * s3).reshape(1, _DH)

    xp = jnp.pad(x, ((0, _NPAD - _N), (0, 0)))
    h = _tc_mlp(xp, w1f, b1f, w2f, b2f, w3f, b3f, W4, b4.reshape(1, _F))

    loops = jnp.arange(_N, dtype=edge_index.dtype)
    padn = _ETPAD - _ET
    prow = _TRASH + (jnp.arange(padn, dtype=edge_index.dtype) % (_NPAD - _N))
    pcol = jnp.arange(padn, dtype=edge_index.dtype) % _N
    row = jnp.concatenate([edge_index[0], loops, prow])
    col = jnp.concatenate([edge_index[1], loops, pcol])
    row3 = row.reshape(_NW, _CHUNKS, _C)
    col3 = col.reshape(_NW, _CHUNKS, _C)

    avecs = jnp.broadcast_to((alpha / _TEMP)[:, None], (_KHOP + 1, _L))

    degp = _sc_deg(row3)
    dinv64, w, xprop = _sc_init(degp, h, avecs[0])
    for k in range(1, _KHOP + 1):
        p = _sc_hop(w, col3, row3)
        w, xprop = _sc_combine(p, dinv64, xprop, avecs[k])
    return xprop[:_N]
